# baseline (speedup 1.0000x reference)
import jax
import jax.numpy as jnp
from jax import lax
from jax.experimental import pallas as pl
from jax.experimental.pallas import tpu as pltpu
from jax.experimental.pallas import tpu_sc as plsc

_MAX_REL = 128
_SQ = 2048
_TQ = 256
_TK = 256
_L = 512
_ND = 15
_SHIFT = _SQ - _MAX_REL - 1


def _sc_windows(t_hbm, f_hbm, t_v, row_v):
    wid = lax.axis_index("s") * 2 + lax.axis_index("c")

    @pl.when(wid < _ND)
    def _():
        pltpu.sync_copy(t_hbm, t_v)
        lane = lax.iota(jnp.int32, 16)

        def chunk(c, carry):
            idx = jnp.clip(256 * wid + 16 * c + lane - _SHIFT, 0, 2 * _MAX_REL)
            row_v[pl.ds(16 * c, 16)] = plsc.load_gather(t_v, [idx])
            return carry

        lax.fori_loop(0, _L // 16, chunk, 0)
        pltpu.sync_copy(row_v, f_hbm.at[wid, 0])


def _build_windows(relative_biases):
    mesh = plsc.VectorSubcoreMesh(core_axis_name="c", subcore_axis_name="s")
    cp = pltpu.CompilerParams(needs_layout_passes=False)
    return pl.kernel(
        _sc_windows,
        mesh=mesh,
        compiler_params=cp,
        out_type=jax.ShapeDtypeStruct((_ND, 1, _L), jnp.float32),
        scratch_types=[
            pltpu.VMEM((257,), jnp.float32),
            pltpu.VMEM((_L,), jnp.float32),
        ],
    )(relative_biases)


def _add_bias_body(f_ref, x_ref, o_ref):
    f = f_ref[0, 0, :]
    fb = jnp.broadcast_to(f[None, :], (_TQ, _L))
    bias = pltpu.roll(fb, _L - _TQ + 1, axis=1, stride=1, stride_axis=0)
    o_ref[...] = x_ref[...] + bias[None, :, :_TK]


def kernel(inputs, relative_biases):
    f_all = _build_windows(relative_biases)

    b = inputs.shape[0]
    grid = (_SQ // _TQ, _SQ // _TK)

    return pl.pallas_call(
        _add_bias_body,
        grid=grid,
        in_specs=[
            pl.BlockSpec((1, 1, _L), lambda qi, ki: (ki - qi + 7, 0, 0)),
            pl.BlockSpec((b, _TQ, _TK), lambda qi, ki: (0, qi, ki)),
        ],
        out_specs=pl.BlockSpec((b, _TQ, _TK), lambda qi, ki: (0, qi, ki)),
        out_shape=jax.ShapeDtypeStruct(inputs.shape, inputs.dtype),
    )(f_all, inputs)
